# baseline (speedup 1.0000x reference)
import functools

import jax
import jax.numpy as jnp
from jax import lax
from jax.experimental import pallas as pl
from jax.experimental.pallas import tpu as pltpu
from jax.experimental.pallas import tpu_sc as plsc

N_NODES = 10000
NC = 2
NS = 16
NW = NC * NS
BLK = 125
RPT = 640
N_PAD = NS * RPT

_MESH = dict(core_axis_name="c", subcore_axis_name="s")
_SC_PARAMS = dict(
    mesh=plsc.VectorSubcoreMesh(**_MESH),
    compiler_params=pltpu.CompilerParams(needs_layout_passes=False,
                                         use_tc_tiling_on_sc=False),
)


def _sc_degree(edge_index):
    ept = edge_index.shape[1] // NW

    @functools.partial(
        pl.kernel,
        out_type=jax.ShapeDtypeStruct((NC, NS, N_NODES), jnp.float32),
        scratch_types=[
            pltpu.VMEM((ept,), jnp.int32),
            pltpu.VMEM((N_NODES,), jnp.float32),
        ],
        **_SC_PARAMS,
    )
    def deg_kernel(ei_hbm, out_hbm, idx_v, deg_v):
        c = lax.axis_index("c")
        s = lax.axis_index("s")
        wid = c * NS + s
        pltpu.sync_copy(ei_hbm.at[1, pl.ds(wid * ept, ept)], idx_v)

        zeros = jnp.zeros((16,), jnp.float32)

        def zero_body(i, carry):
            deg_v[pl.ds(i * 16, 16)] = zeros
            return carry

        lax.fori_loop(0, N_NODES // 16, zero_body, 0)

        ones = jnp.ones((16,), jnp.float32)

        def add_body(i, carry):
            idx = idx_v[pl.ds(i * 16, 16)]
            plsc.addupdate_scatter(deg_v, [idx], ones)
            return carry

        lax.fori_loop(0, ept // 16, add_body, 0)
        pltpu.sync_copy(deg_v, out_hbm.at[c, s])

    return deg_kernel(edge_index)


_NSLOT = 4


def _edge_loop(g_hbm, acc, sidx, didx, rows, gsems, ssems, nblk):
    for b in range(_NSLOT):
        pltpu.async_copy(g_hbm.at[sidx.at[b]], rows.at[b], gsems[b])

    def body(i, carry):
        j = i * _NSLOT
        for b in range(_NSLOT):
            pltpu.make_async_copy(g_hbm.at[sidx.at[j + b]], rows.at[b],
                                  gsems[b]).wait()
            pltpu.async_copy(rows.at[b], acc.at[didx.at[j + b]], ssems[b],
                             add=True)
        for b in range(_NSLOT):
            pltpu.make_async_copy(rows.at[b], acc.at[didx.at[j + b]],
                                  ssems[b]).wait()

            @pl.when(j + b + _NSLOT < nblk)
            def _():
                pltpu.async_copy(g_hbm.at[sidx.at[j + b + _NSLOT]],
                                 rows.at[b], gsems[b])

        return carry

    lax.fori_loop(0, nblk // _NSLOT, body, 0)


def _scatter_scratch(nblk, d):
    return [
        pltpu.VMEM((nblk, BLK), jnp.int32),
        pltpu.VMEM((nblk, BLK), jnp.int32),
        pltpu.VMEM((_NSLOT, BLK, d), jnp.float32),
        pltpu.VMEM_SHARED((N_PAD, d), jnp.float32),
    ] + [pltpu.SemaphoreType.DMA] * (2 * _NSLOT)


def _sc_scatter_fsplit(g_flat, src_off, dst_r, zrows):
    d = g_flat.shape[1]
    nblk = dst_r.shape[1]

    @functools.partial(
        pl.kernel,
        out_type=jax.ShapeDtypeStruct((N_PAD, NC * d), jnp.float32),
        scratch_types=_scatter_scratch(nblk, d),
        **_SC_PARAMS,
    )
    def scatter_kernel(g_hbm, src_hbm, dst_hbm, z_hbm, out_hbm,
                       sidx, didx, rows, acc, *sems):
        c = lax.axis_index("c")
        s = lax.axis_index("s")
        pltpu.sync_copy(src_hbm.at[c, s], sidx)
        pltpu.sync_copy(dst_hbm.at[s], didx)
        pltpu.sync_copy(z_hbm, acc.at[pl.ds(s * RPT, RPT)])
        plsc.subcore_barrier()
        _edge_loop(g_hbm, acc, sidx, didx, rows,
                   sems[:_NSLOT], sems[_NSLOT:], nblk)
        plsc.subcore_barrier()
        pltpu.sync_copy(acc.at[pl.ds(s * RPT, RPT)],
                        out_hbm.at[pl.ds(s * RPT, RPT), pl.ds(c * d, d)])

    return scatter_kernel(g_flat, src_off, dst_r, zrows)


def _sc_scatter_esplit(g, src_r, dst_r, zrows):
    d = g.shape[1]
    nblk = src_r.shape[2]

    @functools.partial(
        pl.kernel,
        out_type=jax.ShapeDtypeStruct((NC, N_PAD, d), jnp.float32),
        scratch_types=_scatter_scratch(nblk, d),
        **_SC_PARAMS,
    )
    def scatter_kernel(g_hbm, src_hbm, dst_hbm, z_hbm, out_hbm,
                       sidx, didx, rows, acc, *sems):
        c = lax.axis_index("c")
        s = lax.axis_index("s")
        pltpu.sync_copy(src_hbm.at[c, s], sidx)
        pltpu.sync_copy(dst_hbm.at[c, s], didx)
        pltpu.sync_copy(z_hbm, acc.at[pl.ds(s * RPT, RPT)])
        plsc.subcore_barrier()
        _edge_loop(g_hbm, acc, sidx, didx, rows,
                   sems[:_NSLOT], sems[_NSLOT:], nblk)
        plsc.subcore_barrier()
        pltpu.sync_copy(acc.at[pl.ds(s * RPT, RPT)],
                        out_hbm.at[c, pl.ds(s * RPT, RPT)])

    return scatter_kernel(g, src_r, dst_r, zrows)


def _tc_dinv(deg_parts):
    def dinv_kernel(deg_ref, out_ref):
        deg = jnp.sum(deg_ref[...], axis=0) + 1.0
        out_ref[...] = lax.rsqrt(deg)[:, None]

    return pl.pallas_call(
        dinv_kernel,
        out_shape=jax.ShapeDtypeStruct((N_NODES, 1), jnp.float32),
    )(deg_parts)


_BS = 2000


def _tc_matmul(x, w1):
    f_in, f_out = w1.shape

    def mm_kernel(x_ref, w_ref, h_ref):
        h_ref[...] = jnp.dot(x_ref[...], w_ref[...],
                             preferred_element_type=jnp.float32)

    return pl.pallas_call(
        mm_kernel,
        grid=(N_NODES // _BS,),
        in_specs=[
            pl.BlockSpec((_BS, f_in), lambda i: (i, 0)),
            pl.BlockSpec((f_in, f_out), lambda i: (0, 0)),
        ],
        out_specs=pl.BlockSpec((_BS, f_out), lambda i: (i, 0)),
        out_shape=jax.ShapeDtypeStruct((N_NODES, f_out), jnp.float32),
    )(x, w1)


def _tc_scale(h1, dinv):
    f_out = h1.shape[1]

    def sc_kernel(h_ref, dv_ref, g_ref):
        g_ref[...] = h_ref[...] * dv_ref[...]

    return pl.pallas_call(
        sc_kernel,
        grid=(N_NODES // _BS,),
        in_specs=[
            pl.BlockSpec((_BS, f_out), lambda i: (i, 0)),
            pl.BlockSpec((_BS, 1), lambda i: (i, 0)),
        ],
        out_specs=pl.BlockSpec((_BS, f_out), lambda i: (i, 0)),
        out_shape=jax.ShapeDtypeStruct((N_NODES, f_out), jnp.float32),
    )(h1, dinv)


def _tc_layer2(agg1, h1, b1, w2, dinv):
    f_in, f_out = w2.shape

    def l2_kernel(a_ref, h1_ref, b1_ref, w2_ref, dv_ref, h2_ref, g2_ref):
        dv = dv_ref[...]
        o = a_ref[...] * dv + h1_ref[...] * (dv * dv) + b1_ref[...]
        r = jnp.maximum(o, 0.0)
        h2 = jnp.dot(r, w2_ref[...], preferred_element_type=jnp.float32)
        h2_ref[...] = h2
        g2_ref[...] = h2 * dv

    return pl.pallas_call(
        l2_kernel,
        grid=(N_NODES // _BS,),
        in_specs=[
            pl.BlockSpec((_BS, f_in), lambda i: (i, 0)),
            pl.BlockSpec((_BS, f_in), lambda i: (i, 0)),
            pl.BlockSpec((1, f_in), lambda i: (0, 0)),
            pl.BlockSpec((f_in, f_out), lambda i: (0, 0)),
            pl.BlockSpec((_BS, 1), lambda i: (i, 0)),
        ],
        out_specs=[
            pl.BlockSpec((_BS, f_out), lambda i: (i, 0)),
            pl.BlockSpec((_BS, f_out), lambda i: (i, 0)),
        ],
        out_shape=[
            jax.ShapeDtypeStruct((N_NODES, f_out), jnp.float32),
            jax.ShapeDtypeStruct((N_NODES, f_out), jnp.float32),
        ],
    )(agg1, h1, b1, w2, dinv)


def _tc_out(agg2, h2, b2, dinv):
    f_out = h2.shape[1]

    def out_kernel(a_ref, h2_ref, b2_ref, dv_ref, o_ref):
        dv = dv_ref[...]
        o = (a_ref[0] + a_ref[1]) * dv + h2_ref[...] * (dv * dv) + b2_ref[...]
        m = jnp.max(o, axis=1, keepdims=True)
        lse = jnp.log(jnp.sum(jnp.exp(o - m), axis=1, keepdims=True)) + m
        o_ref[...] = o - lse

    return pl.pallas_call(
        out_kernel,
        grid=(N_NODES // _BS,),
        in_specs=[
            pl.BlockSpec((NC, _BS, f_out), lambda i: (0, i, 0)),
            pl.BlockSpec((_BS, f_out), lambda i: (i, 0)),
            pl.BlockSpec((1, f_out), lambda i: (0, 0)),
            pl.BlockSpec((_BS, 1), lambda i: (i, 0)),
        ],
        out_specs=pl.BlockSpec((_BS, f_out), lambda i: (i, 0)),
        out_shape=jax.ShapeDtypeStruct((N_NODES, f_out), jnp.float32),
    )(agg2, h2, b2, dinv)


def kernel(x, edge_index, W1, b1, W2, b2):
    n_edges = edge_index.shape[1]
    ei32 = edge_index.astype(jnp.int32)
    src = ei32[0]
    dst = ei32[1]

    ept2 = n_edges // NW
    nblk2 = ept2 // BLK
    assert nblk2 * BLK * NW == n_edges and nblk2 % 2 == 0
    src_e = src.reshape(NC, NS, nblk2, BLK)
    dst_e = dst.reshape(NC, NS, nblk2, BLK)

    ept1 = n_edges // NS
    nblk1 = ept1 // BLK
    assert nblk1 % _NSLOT == 0 and (n_edges // NW) // BLK % _NSLOT == 0
    src_f = src.reshape(NS, nblk1, BLK)
    src_off = jnp.stack([2 * src_f, 2 * src_f + 1])
    dst_f = dst.reshape(NS, nblk1, BLK)

    deg_parts = _sc_degree(ei32)
    h1 = _tc_matmul(x, W1)
    dinv = _tc_dinv(deg_parts.reshape(NW, N_NODES))

    g1 = _tc_scale(h1, dinv)
    half = W1.shape[1] // NC
    g1_flat = g1.reshape(NC * N_NODES, half)
    agg1 = _sc_scatter_fsplit(g1_flat, src_off, dst_f,
                              jnp.zeros((RPT, half), jnp.float32))
    h2, g2 = _tc_layer2(agg1, h1, b1.reshape(1, -1), W2, dinv)
    agg2 = _sc_scatter_esplit(g2, src_e, dst_e,
                              jnp.zeros((RPT, W2.shape[1]), jnp.float32))
    return _tc_out(agg2, h2, b2.reshape(1, -1), dinv)


# speedup vs baseline: 1.0400x; 1.0400x over previous
import functools

import jax
import jax.numpy as jnp
from jax import lax
from jax.experimental import pallas as pl
from jax.experimental.pallas import tpu as pltpu
from jax.experimental.pallas import tpu_sc as plsc

N_NODES = 10000
NC = 2
NS = 16
NW = NC * NS
BLK = 125
RPT = 640
N_PAD = NS * RPT

_MESH = dict(core_axis_name="c", subcore_axis_name="s")
_SC_PARAMS = dict(
    mesh=plsc.VectorSubcoreMesh(**_MESH),
    compiler_params=pltpu.CompilerParams(needs_layout_passes=False,
                                         use_tc_tiling_on_sc=False),
)


def _sc_degree(edge_index):
    ept = edge_index.shape[1] // NW

    @functools.partial(
        pl.kernel,
        out_type=jax.ShapeDtypeStruct((NC, NS, N_NODES), jnp.float32),
        scratch_types=[
            pltpu.VMEM((ept,), jnp.int32),
            pltpu.VMEM((N_NODES,), jnp.float32),
        ],
        **_SC_PARAMS,
    )
    def deg_kernel(ei_hbm, out_hbm, idx_v, deg_v):
        c = lax.axis_index("c")
        s = lax.axis_index("s")
        wid = c * NS + s
        pltpu.sync_copy(ei_hbm.at[1, pl.ds(wid * ept, ept)], idx_v)

        zeros = jnp.zeros((16,), jnp.float32)

        def zero_body(i, carry):
            deg_v[pl.ds(i * 16, 16)] = zeros
            return carry

        lax.fori_loop(0, N_NODES // 16, zero_body, 0)

        ones = jnp.ones((16,), jnp.float32)

        def add_body(i, carry):
            idx = idx_v[pl.ds(i * 16, 16)]
            plsc.addupdate_scatter(deg_v, [idx], ones)
            return carry

        lax.fori_loop(0, ept // 16, add_body, 0)
        pltpu.sync_copy(deg_v, out_hbm.at[c, s])

    return deg_kernel(edge_index)


_NSLOT = 5


def _edge_loop(g_hbm, acc, sidx, didx, rows, gsems, ssems, nblk):
    for b in range(_NSLOT):
        pltpu.async_copy(g_hbm.at[sidx.at[b]], rows.at[b], gsems[b])

    def body(i, carry):
        j = i * _NSLOT
        for b in range(_NSLOT):
            pltpu.make_async_copy(g_hbm.at[sidx.at[j + b]], rows.at[b],
                                  gsems[b]).wait()
            pltpu.async_copy(rows.at[b], acc.at[didx.at[j + b]], ssems[b],
                             add=True)
        for b in range(_NSLOT):
            pltpu.make_async_copy(rows.at[b], acc.at[didx.at[j + b]],
                                  ssems[b]).wait()

            @pl.when(j + b + _NSLOT < nblk)
            def _():
                pltpu.async_copy(g_hbm.at[sidx.at[j + b + _NSLOT]],
                                 rows.at[b], gsems[b])

        return carry

    lax.fori_loop(0, nblk // _NSLOT, body, 0)


def _scatter_scratch(nblk, d):
    return [
        pltpu.VMEM((nblk, BLK), jnp.int32),
        pltpu.VMEM((nblk, BLK), jnp.int32),
        pltpu.VMEM((_NSLOT, BLK, d), jnp.float32),
        pltpu.VMEM_SHARED((N_PAD, d), jnp.float32),
    ] + [pltpu.SemaphoreType.DMA] * (2 * _NSLOT)


def _sc_scatter_fsplit(g_flat, src_off, dst_r, zrows):
    d = g_flat.shape[1]
    nblk = dst_r.shape[1]

    @functools.partial(
        pl.kernel,
        out_type=jax.ShapeDtypeStruct((N_PAD, NC * d), jnp.float32),
        scratch_types=_scatter_scratch(nblk, d),
        **_SC_PARAMS,
    )
    def scatter_kernel(g_hbm, src_hbm, dst_hbm, z_hbm, out_hbm,
                       sidx, didx, rows, acc, *sems):
        c = lax.axis_index("c")
        s = lax.axis_index("s")
        pltpu.sync_copy(src_hbm.at[c, s], sidx)
        pltpu.sync_copy(dst_hbm.at[s], didx)
        pltpu.sync_copy(z_hbm, acc.at[pl.ds(s * RPT, RPT)])
        plsc.subcore_barrier()
        _edge_loop(g_hbm, acc, sidx, didx, rows,
                   sems[:_NSLOT], sems[_NSLOT:], nblk)
        plsc.subcore_barrier()
        pltpu.sync_copy(acc.at[pl.ds(s * RPT, RPT)],
                        out_hbm.at[pl.ds(s * RPT, RPT), pl.ds(c * d, d)])

    return scatter_kernel(g_flat, src_off, dst_r, zrows)


def _sc_scatter_esplit(g, src_r, dst_r, zrows):
    d = g.shape[1]
    nblk = src_r.shape[2]

    @functools.partial(
        pl.kernel,
        out_type=jax.ShapeDtypeStruct((NC, N_PAD, d), jnp.float32),
        scratch_types=_scatter_scratch(nblk, d),
        **_SC_PARAMS,
    )
    def scatter_kernel(g_hbm, src_hbm, dst_hbm, z_hbm, out_hbm,
                       sidx, didx, rows, acc, *sems):
        c = lax.axis_index("c")
        s = lax.axis_index("s")
        pltpu.sync_copy(src_hbm.at[c, s], sidx)
        pltpu.sync_copy(dst_hbm.at[c, s], didx)
        pltpu.sync_copy(z_hbm, acc.at[pl.ds(s * RPT, RPT)])
        plsc.subcore_barrier()
        _edge_loop(g_hbm, acc, sidx, didx, rows,
                   sems[:_NSLOT], sems[_NSLOT:], nblk)
        plsc.subcore_barrier()
        pltpu.sync_copy(acc.at[pl.ds(s * RPT, RPT)],
                        out_hbm.at[c, pl.ds(s * RPT, RPT)])

    return scatter_kernel(g, src_r, dst_r, zrows)


def _tc_dinv(deg_parts):
    def dinv_kernel(deg_ref, out_ref):
        deg = jnp.sum(deg_ref[...], axis=0) + 1.0
        out_ref[...] = lax.rsqrt(deg)[:, None]

    return pl.pallas_call(
        dinv_kernel,
        out_shape=jax.ShapeDtypeStruct((N_NODES, 1), jnp.float32),
    )(deg_parts)


_BS = 2000


def _tc_layer1(x, w1, dinv):
    f_in, f_out = w1.shape

    def l1_kernel(x_ref, w_ref, dv_ref, h_ref, g_ref):
        h = jnp.dot(x_ref[...], w_ref[...], preferred_element_type=jnp.float32)
        h_ref[...] = h
        g_ref[...] = h * dv_ref[...]

    return pl.pallas_call(
        l1_kernel,
        grid=(N_NODES // _BS,),
        in_specs=[
            pl.BlockSpec((_BS, f_in), lambda i: (i, 0)),
            pl.BlockSpec((f_in, f_out), lambda i: (0, 0)),
            pl.BlockSpec((_BS, 1), lambda i: (i, 0)),
        ],
        out_specs=[
            pl.BlockSpec((_BS, f_out), lambda i: (i, 0)),
            pl.BlockSpec((_BS, f_out), lambda i: (i, 0)),
        ],
        out_shape=[
            jax.ShapeDtypeStruct((N_NODES, f_out), jnp.float32),
            jax.ShapeDtypeStruct((N_NODES, f_out), jnp.float32),
        ],
    )(x, w1, dinv)


def _tc_layer2(agg1, h1, b1, w2, dinv):
    f_in, f_out = w2.shape

    def l2_kernel(a_ref, h1_ref, b1_ref, w2_ref, dv_ref, h2_ref, g2_ref):
        dv = dv_ref[...]
        o = a_ref[...] * dv + h1_ref[...] * (dv * dv) + b1_ref[...]
        r = jnp.maximum(o, 0.0)
        h2 = jnp.dot(r, w2_ref[...], preferred_element_type=jnp.float32)
        h2_ref[...] = h2
        g2_ref[...] = h2 * dv

    return pl.pallas_call(
        l2_kernel,
        grid=(N_NODES // _BS,),
        in_specs=[
            pl.BlockSpec((_BS, f_in), lambda i: (i, 0)),
            pl.BlockSpec((_BS, f_in), lambda i: (i, 0)),
            pl.BlockSpec((1, f_in), lambda i: (0, 0)),
            pl.BlockSpec((f_in, f_out), lambda i: (0, 0)),
            pl.BlockSpec((_BS, 1), lambda i: (i, 0)),
        ],
        out_specs=[
            pl.BlockSpec((_BS, f_out), lambda i: (i, 0)),
            pl.BlockSpec((_BS, f_out), lambda i: (i, 0)),
        ],
        out_shape=[
            jax.ShapeDtypeStruct((N_NODES, f_out), jnp.float32),
            jax.ShapeDtypeStruct((N_NODES, f_out), jnp.float32),
        ],
    )(agg1, h1, b1, w2, dinv)


def _tc_out(agg2, h2, b2, dinv):
    f_out = h2.shape[1]

    def out_kernel(a_ref, h2_ref, b2_ref, dv_ref, o_ref):
        dv = dv_ref[...]
        o = (a_ref[0] + a_ref[1]) * dv + h2_ref[...] * (dv * dv) + b2_ref[...]
        m = jnp.max(o, axis=1, keepdims=True)
        lse = jnp.log(jnp.sum(jnp.exp(o - m), axis=1, keepdims=True)) + m
        o_ref[...] = o - lse

    return pl.pallas_call(
        out_kernel,
        grid=(N_NODES // _BS,),
        in_specs=[
            pl.BlockSpec((NC, _BS, f_out), lambda i: (0, i, 0)),
            pl.BlockSpec((_BS, f_out), lambda i: (i, 0)),
            pl.BlockSpec((1, f_out), lambda i: (0, 0)),
            pl.BlockSpec((_BS, 1), lambda i: (i, 0)),
        ],
        out_specs=pl.BlockSpec((_BS, f_out), lambda i: (i, 0)),
        out_shape=jax.ShapeDtypeStruct((N_NODES, f_out), jnp.float32),
    )(agg2, h2, b2, dinv)


def kernel(x, edge_index, W1, b1, W2, b2):
    n_edges = edge_index.shape[1]
    ei32 = edge_index.astype(jnp.int32)
    src = ei32[0]
    dst = ei32[1]

    ept2 = n_edges // NW
    nblk2 = ept2 // BLK
    assert nblk2 * BLK * NW == n_edges and nblk2 % 2 == 0
    src_e = src.reshape(NC, NS, nblk2, BLK)
    dst_e = dst.reshape(NC, NS, nblk2, BLK)

    ept1 = n_edges // NS
    nblk1 = ept1 // BLK
    assert nblk1 % _NSLOT == 0 and (n_edges // NW) // BLK % _NSLOT == 0
    src_f = src.reshape(NS, nblk1, BLK)
    src_off = jnp.stack([2 * src_f, 2 * src_f + 1])
    dst_f = dst.reshape(NS, nblk1, BLK)

    deg_parts = _sc_degree(ei32)
    dinv = _tc_dinv(deg_parts.reshape(NW, N_NODES))

    h1, g1 = _tc_layer1(x, W1, dinv)
    half = W1.shape[1] // NC
    g1_flat = g1.reshape(NC * N_NODES, half)
    agg1 = _sc_scatter_fsplit(g1_flat, src_off, dst_f,
                              jnp.zeros((RPT, half), jnp.float32))
    h2, g2 = _tc_layer2(agg1, h1, b1.reshape(1, -1), W2, dinv)
    agg2 = _sc_scatter_esplit(g2, src_e, dst_e,
                              jnp.zeros((RPT, W2.shape[1]), jnp.float32))
    return _tc_out(agg2, h2, b2.reshape(1, -1), dinv)
